# baseline (speedup 1.0000x reference)
import functools

import jax
import jax.numpy as jnp
from jax import lax
from jax.experimental import pallas as pl
from jax.experimental.pallas import tpu as pltpu
from jax.experimental.pallas import tpu_sc as plsc

_NC = 2
_NS = 16
_NW = _NC * _NS
_LANES = 16


@functools.partial(jax.jit, static_argnames=("batch",))
def _irt_sc(skills, ability16, difficulty, discrimination, *, batch):
    b_per_w = batch // _NW
    mesh = plsc.VectorSubcoreMesh(
        core_axis_name="c", subcore_axis_name="s",
        num_cores=_NC, num_subcores=_NS)

    @functools.partial(
        pl.kernel,
        out_type=jax.ShapeDtypeStruct((batch,), jnp.float32),
        mesh=mesh,
        scratch_types=[
            pltpu.VMEM((b_per_w,), jnp.int32),
            pltpu.VMEM((b_per_w,), jnp.float32),
            pltpu.VMEM((b_per_w,), jnp.float32),
            pltpu.VMEM((_LANES,), jnp.float32),
            pltpu.SemaphoreType.DMA,
        ],
    )
    def k(skills_hbm, ab_hbm, diff_hbm, disc_hbm, out_hbm,
          idx_v, diff_v, disc_v, ab_v, sem):
        wid = lax.axis_index("s") * _NC + lax.axis_index("c")
        base = wid * b_per_w
        pltpu.sync_copy(skills_hbm.at[pl.ds(base, b_per_w)], idx_v)
        cp1 = pltpu.async_copy(diff_hbm.at[idx_v], diff_v, sem)
        cp2 = pltpu.async_copy(disc_hbm.at[idx_v], disc_v, sem)
        pltpu.sync_copy(ab_hbm, ab_v)
        cp1.wait()
        cp2.wait()
        a = ab_v[:]
        for i in range(b_per_w // _LANES):
            sl = pl.ds(i * _LANES, _LANES)
            x = disc_v[sl] * (a - diff_v[sl])
            diff_v[sl] = 1.0 / (1.0 + jnp.exp(-x))
        pltpu.sync_copy(diff_v, out_hbm.at[pl.ds(base, b_per_w)])

    return k(skills, ability16, difficulty, discrimination)


def kernel(skills, ability_table, difficulty_table, discrimination_table):
    batch = skills.shape[0]
    ability16 = jnp.broadcast_to(ability_table.reshape(()), (_LANES,))
    diff = difficulty_table.reshape(-1)
    disc = discrimination_table.reshape(-1)
    out = _irt_sc(skills.astype(jnp.int32), ability16, diff, disc, batch=batch)
    return out.reshape(batch, 1)


# speedup vs baseline: 1.0047x; 1.0047x over previous
import functools

import jax
import jax.numpy as jnp
from jax import lax
from jax.experimental import pallas as pl
from jax.experimental.pallas import tpu as pltpu
from jax.experimental.pallas import tpu_sc as plsc

_NC = 2
_NS = 16
_NW = _NC * _NS
_LANES = 16


@functools.partial(jax.jit, static_argnames=("batch",))
def _irt_sc(skills, ability_table, difficulty, discrimination, *, batch):
    b_per_w = batch // _NW
    half = b_per_w // 2
    mesh = plsc.VectorSubcoreMesh(
        core_axis_name="c", subcore_axis_name="s",
        num_cores=_NC, num_subcores=_NS)

    @functools.partial(
        pl.kernel,
        out_type=jax.ShapeDtypeStruct((batch,), jnp.float32),
        mesh=mesh,
        scratch_types=[
            pltpu.VMEM((b_per_w,), jnp.int32),
            pltpu.VMEM((b_per_w,), jnp.float32),
            pltpu.VMEM((b_per_w,), jnp.float32),
            pltpu.VMEM((_LANES,), jnp.float32),
            pltpu.SemaphoreType.DMA,
            pltpu.SemaphoreType.DMA,
        ],
    )
    def k(skills_hbm, ab_hbm, diff_hbm, disc_hbm, out_hbm,
          idx_v, diff_v, disc_v, ab_v, sem0, sem1):
        wid = lax.axis_index("s") * _NC + lax.axis_index("c")
        base = wid * b_per_w
        lo = pl.ds(0, half)
        hi = pl.ds(half, half)
        pltpu.sync_copy(skills_hbm.at[pl.ds(base, b_per_w)], idx_v)
        cp = [
            pltpu.async_copy(diff_hbm.at[idx_v.at[lo]], diff_v.at[lo], sem0),
            pltpu.async_copy(disc_hbm.at[idx_v.at[lo]], disc_v.at[lo], sem0),
            pltpu.async_copy(diff_hbm.at[idx_v.at[hi]], diff_v.at[hi], sem1),
            pltpu.async_copy(disc_hbm.at[idx_v.at[hi]], disc_v.at[hi], sem1),
        ]
        pltpu.sync_copy(ab_hbm, ab_v)
        a = ab_v[:]
        cp[0].wait()
        cp[1].wait()
        for i in range(half // _LANES):
            sl = pl.ds(i * _LANES, _LANES)
            x = disc_v[sl] * (a - diff_v[sl])
            diff_v[sl] = 1.0 / (1.0 + jnp.exp(-x))
        cp[2].wait()
        cp[3].wait()
        for i in range(half // _LANES, b_per_w // _LANES):
            sl = pl.ds(i * _LANES, _LANES)
            x = disc_v[sl] * (a - diff_v[sl])
            diff_v[sl] = 1.0 / (1.0 + jnp.exp(-x))
        pltpu.sync_copy(diff_v, out_hbm.at[pl.ds(base, b_per_w)])

    return k(skills, ability_table, difficulty, discrimination)


def kernel(skills, ability_table, difficulty_table, discrimination_table):
    batch = skills.shape[0]
    if skills.dtype != jnp.int32:
        skills = skills.astype(jnp.int32)
    diff = difficulty_table.reshape(-1)
    disc = discrimination_table.reshape(-1)
    ability16 = jnp.broadcast_to(ability_table.reshape(()), (_LANES,))
    out = _irt_sc(skills, ability16, diff, disc, batch=batch)
    return out.reshape(batch, 1)
